# baseline (speedup 1.0000x reference)
import jax
import jax.numpy as jnp
from jax.experimental import pallas as pl
from jax.experimental.pallas import tpu as pltpu

NUM_EXPERTS = 8
TOP_K = 2
D_MODEL = 1024
D_FF = 512
T_TOK = 2048
TB = 256
NTB = T_TOK // TB


def _dot(a, b):
    return jax.lax.dot_general(
        a, b, (((1,), (0,)), ((), ())), preferred_element_type=jnp.float32
    )


def _moe_kernel(x_ref, gate_ref, sw1_ref, sb1_ref, sw2_ref, sb2_ref,
                sgw_ref, sgb_ref, w1_ref, b1_ref, w2_ref, b2_ref,
                out_ref, comb_ref, sh_ref):
    s = pl.program_id(0)

    @pl.when(s == 0)
    def _():
        x = x_ref[...]
        logits = _dot(x, gate_ref[...])
        m = jnp.max(logits, axis=-1, keepdims=True)
        ee = jnp.exp(logits - m)
        probs = ee / jnp.sum(ee, axis=-1, keepdims=True)

        iota = jax.lax.broadcasted_iota(jnp.int32, probs.shape, 1)
        w1 = jnp.max(probs, axis=-1, keepdims=True)
        is1 = probs == w1
        i1 = jnp.min(jnp.where(is1, iota, NUM_EXPERTS), axis=-1, keepdims=True)
        mask1 = iota == i1
        probs2 = jnp.where(mask1, -jnp.inf, probs)
        w2 = jnp.max(probs2, axis=-1, keepdims=True)
        is2 = probs2 == w2
        i2 = jnp.min(jnp.where(is2, iota, NUM_EXPERTS), axis=-1, keepdims=True)
        mask2 = iota == i2
        comb_ref[...] = jnp.where(mask1 | mask2, probs, 0.0) / (w1 + w2)

        sw2_bf = sw2_ref[...].astype(jnp.bfloat16)
        for cc in range(NTB):
            xc = x[cc * TB:(cc + 1) * TB]
            hs = _dot(xc, sw1_ref[...]) + sb1_ref[...]
            hs = jax.nn.gelu(hs.astype(jnp.bfloat16))
            ys = _dot(hs, sw2_bf) + sb2_ref[...]
            g = jax.nn.sigmoid(_dot(xc, sgw_ref[...]) + sgb_ref[...])
            sh_ref[pl.ds(cc * TB, TB), :] = (g * ys).astype(jnp.bfloat16)

    @pl.when(s > 0)
    def _():
        i = s - 1
        xb = x_ref[pl.ds(i * TB, TB), :]
        comb = comb_ref[pl.ds(i * TB, TB), :]
        acc = sh_ref[pl.ds(i * TB, TB), :].astype(jnp.float32)
        for ex in range(NUM_EXPERTS):
            h = _dot(xb, w1_ref[ex]) + b1_ref[ex][None, :]
            h = jax.nn.gelu(h.astype(jnp.bfloat16))
            y = _dot(h, w2_ref[ex].astype(jnp.bfloat16)) + b2_ref[ex][None, :]
            acc = acc + comb[:, ex:ex + 1] * y
        out_ref[...] = acc


@jax.jit
def kernel(hidden_states, gate_w, W1, b1, W2, b2, shared_W1, shared_b1,
           shared_W2, shared_b2, sgate_w, sgate_b):
    T, D = hidden_states.shape

    sb1_2d = shared_b1.reshape(1, D_FF)
    sb2_2d = shared_b2.reshape(1, D_MODEL)
    sgb_2d = sgate_b.reshape(1, 1)

    full = lambda *shape: pl.BlockSpec(shape, lambda s: (0,) * len(shape))
    out = pl.pallas_call(
        _moe_kernel,
        grid=(1 + NTB,),
        in_specs=[
            full(T, D),
            full(D, NUM_EXPERTS),
            full(D, D_FF),
            full(1, D_FF),
            full(D_FF, D),
            full(1, D),
            full(D, 1),
            full(1, 1),
            full(NUM_EXPERTS, D, D_FF),
            full(NUM_EXPERTS, D_FF),
            full(NUM_EXPERTS, D_FF, D),
            full(NUM_EXPERTS, D),
        ],
        out_specs=pl.BlockSpec(
            (TB, D), lambda s: (jnp.maximum(s - 1, 0), 0)),
        out_shape=jax.ShapeDtypeStruct((T, D), jnp.float32),
        scratch_shapes=[
            pltpu.VMEM((T_TOK, NUM_EXPERTS), jnp.float32),
            pltpu.VMEM((T_TOK, D_MODEL), jnp.bfloat16),
        ],
    )(hidden_states, gate_w, shared_W1, sb1_2d, shared_W2, sb2_2d,
      sgate_w, sgb_2d, W1, b1, W2, b2)
    return out


# speedup vs baseline: 1.2378x; 1.2378x over previous
import jax
import jax.numpy as jnp
from jax.experimental import pallas as pl
from jax.experimental.pallas import tpu as pltpu

NUM_EXPERTS = 8
TOP_K = 2
D_MODEL = 1024
D_FF = 512
TB = 512


def _dot(a, b):
    return jax.lax.dot_general(
        a, b, (((1,), (0,)), ((), ())), preferred_element_type=jnp.float32
    )


def _moe_block_kernel(x_ref, gate_ref, w1_hbm, b1_ref, w2_hbm, b2_ref,
                      sw1_ref, sb1_ref, sw2_ref, sb2_ref, sgw_ref, sgb_ref,
                      out_ref, w1s, w2s, sem):
    s = pl.program_id(0)

    @pl.when(s == 0)
    def _():
        for ex in range(NUM_EXPERTS):
            pltpu.make_async_copy(w1_hbm.at[ex], w1s.at[ex], sem).start()
            pltpu.make_async_copy(w2_hbm.at[ex], w2s.at[ex], sem).start()

    x = x_ref[...]

    logits = _dot(x, gate_ref[...])
    m = jnp.max(logits, axis=-1, keepdims=True)
    e = jnp.exp(logits - m)
    probs = e / jnp.sum(e, axis=-1, keepdims=True)

    iota = jax.lax.broadcasted_iota(jnp.int32, probs.shape, 1)
    w1 = jnp.max(probs, axis=-1, keepdims=True)
    is_max = probs == w1
    i1 = jnp.min(jnp.where(is_max, iota, NUM_EXPERTS), axis=-1, keepdims=True)
    mask1 = iota == i1
    probs2 = jnp.where(mask1, -jnp.inf, probs)
    w2 = jnp.max(probs2, axis=-1, keepdims=True)
    is_max2 = probs2 == w2
    i2 = jnp.min(jnp.where(is_max2, iota, NUM_EXPERTS), axis=-1, keepdims=True)
    mask2 = iota == i2
    denom = w1 + w2
    combine = jnp.where(mask1 | mask2, probs, 0.0) / denom

    hs = _dot(x, sw1_ref[...]) + sb1_ref[...]
    hs = jax.nn.gelu(hs.astype(jnp.bfloat16))
    ys = _dot(hs, sw2_ref[...].astype(jnp.bfloat16)) + sb2_ref[...]
    glog = _dot(x, sgw_ref[...]) + sgb_ref[...]
    g = jax.nn.sigmoid(glog)
    acc = g * ys

    for ex in range(NUM_EXPERTS):
        @pl.when(s == 0)
        def _():
            pltpu.make_async_copy(w1_hbm.at[ex], w1s.at[ex], sem).wait()
            pltpu.make_async_copy(w2_hbm.at[ex], w2s.at[ex], sem).wait()

        h = _dot(x, w1s[ex]) + b1_ref[ex][None, :]
        h = jax.nn.gelu(h.astype(jnp.bfloat16))
        y = _dot(h, w2s[ex].astype(jnp.bfloat16)) + b2_ref[ex][None, :]
        acc = acc + combine[:, ex:ex + 1] * y

    out_ref[...] = acc


@jax.jit
def kernel(hidden_states, gate_w, W1, b1, W2, b2, shared_W1, shared_b1,
           shared_W2, shared_b2, sgate_w, sgate_b):
    T, D = hidden_states.shape
    num_blocks = T // TB

    sb1_2d = shared_b1.reshape(1, D_FF)
    sb2_2d = shared_b2.reshape(1, D_MODEL)
    sgb_2d = sgate_b.reshape(1, 1)

    full = lambda *shape: pl.BlockSpec(shape, lambda i: (0,) * len(shape))
    out = pl.pallas_call(
        _moe_block_kernel,
        grid=(num_blocks,),
        in_specs=[
            pl.BlockSpec((TB, D), lambda i: (i, 0)),
            full(D, NUM_EXPERTS),
            pl.BlockSpec(memory_space=pl.ANY),
            full(NUM_EXPERTS, D_FF),
            pl.BlockSpec(memory_space=pl.ANY),
            full(NUM_EXPERTS, D),
            full(D, D_FF),
            full(1, D_FF),
            full(D_FF, D),
            full(1, D),
            full(D, 1),
            full(1, 1),
        ],
        out_specs=pl.BlockSpec((TB, D), lambda i: (i, 0)),
        out_shape=jax.ShapeDtypeStruct((T, D), jnp.float32),
        scratch_shapes=[
            pltpu.VMEM((NUM_EXPERTS, D_MODEL, D_FF), jnp.float32),
            pltpu.VMEM((NUM_EXPERTS, D_FF, D_MODEL), jnp.float32),
            pltpu.SemaphoreType.DMA,
        ],
    )(hidden_states, gate_w, W1, b1, W2, b2, shared_W1, sb1_2d,
      shared_W2, sb2_2d, sgate_w, sgb_2d)
    return out


# speedup vs baseline: 1.2492x; 1.0092x over previous
import jax
import jax.numpy as jnp
from jax.experimental import pallas as pl

NUM_EXPERTS = 8
TOP_K = 2
D_MODEL = 1024
D_FF = 512
TB = 512


def _dot(a, b):
    return jax.lax.dot_general(
        a, b, (((1,), (0,)), ((), ())), preferred_element_type=jnp.float32
    )


def _moe_block_kernel(x_ref, gate_ref, w1_ref, b1_ref, w2_ref, b2_ref,
                      sw1_ref, sb1_ref, sw2_ref, sb2_ref, sgw_ref, sgb_ref,
                      out_ref):
    x = x_ref[...]

    logits = _dot(x, gate_ref[...])
    m = jnp.max(logits, axis=-1, keepdims=True)
    e = jnp.exp(logits - m)
    probs = e / jnp.sum(e, axis=-1, keepdims=True)

    iota = jax.lax.broadcasted_iota(jnp.int32, probs.shape, 1)
    w1 = jnp.max(probs, axis=-1, keepdims=True)
    is_max = probs == w1
    i1 = jnp.min(jnp.where(is_max, iota, NUM_EXPERTS), axis=-1, keepdims=True)
    mask1 = iota == i1
    probs2 = jnp.where(mask1, -jnp.inf, probs)
    w2 = jnp.max(probs2, axis=-1, keepdims=True)
    is_max2 = probs2 == w2
    i2 = jnp.min(jnp.where(is_max2, iota, NUM_EXPERTS), axis=-1, keepdims=True)
    mask2 = iota == i2
    denom = w1 + w2
    combine = jnp.where(mask1 | mask2, probs, 0.0) / denom

    acc = jnp.zeros((TB, D_MODEL), jnp.float32)
    for ex in range(NUM_EXPERTS):
        h = _dot(x, w1_ref[ex]) + b1_ref[ex][None, :]
        h = jax.nn.gelu(h.astype(jnp.bfloat16))
        y = _dot(h, w2_ref[ex].astype(jnp.bfloat16)) + b2_ref[ex][None, :]
        acc = acc + combine[:, ex:ex + 1] * y

    hs = _dot(x, sw1_ref[...]) + sb1_ref[...]
    hs = jax.nn.gelu(hs.astype(jnp.bfloat16))
    ys = _dot(hs, sw2_ref[...].astype(jnp.bfloat16)) + sb2_ref[...]
    glog = _dot(x, sgw_ref[...]) + sgb_ref[...]
    g = jax.nn.sigmoid(glog)
    out_ref[...] = acc + g * ys


@jax.jit
def kernel(hidden_states, gate_w, W1, b1, W2, b2, shared_W1, shared_b1,
           shared_W2, shared_b2, sgate_w, sgate_b):
    T, D = hidden_states.shape
    num_blocks = T // TB

    sb1_2d = shared_b1.reshape(1, D_FF)
    sb2_2d = shared_b2.reshape(1, D_MODEL)
    sgb_2d = sgate_b.reshape(1, 1)

    full = lambda *shape: pl.BlockSpec(shape, lambda i: (0,) * len(shape))
    out = pl.pallas_call(
        _moe_block_kernel,
        grid=(num_blocks,),
        in_specs=[
            pl.BlockSpec((TB, D), lambda i: (i, 0)),
            full(D, NUM_EXPERTS),
            full(NUM_EXPERTS, D, D_FF),
            full(NUM_EXPERTS, D_FF),
            full(NUM_EXPERTS, D_FF, D),
            full(NUM_EXPERTS, D),
            full(D, D_FF),
            full(1, D_FF),
            full(D_FF, D),
            full(1, D),
            full(D, 1),
            full(1, 1),
        ],
        out_specs=pl.BlockSpec((TB, D), lambda i: (i, 0)),
        out_shape=jax.ShapeDtypeStruct((T, D), jnp.float32),
    )(hidden_states, gate_w, W1, b1, W2, b2, shared_W1, sb1_2d,
      shared_W2, sb2_2d, sgate_w, sgb_2d)
    return out
